# baseline (speedup 1.0000x reference)
import functools

import jax
import jax.numpy as jnp
from jax import lax
from jax.experimental import pallas as pl
from jax.experimental.pallas import tpu as pltpu
from jax.experimental.pallas import tpu_sc as plsc

_E = 800000
_F = 3
_DIM = 32
_N = _E * _F
_NC = 2
_NS = 16
_NW = _NC * _NS
_M = 125
_K = 8
_C = _K * _M
_PER_W = _N // _NW
_STEPS = _PER_W // _C
_IDX_ROWS_W = _PER_W // _M
_REFRESH = 15
_IB = _REFRESH * _K


@functools.partial(
    pl.kernel,
    out_type=jax.ShapeDtypeStruct((_N, _DIM), jnp.float32),
    mesh=plsc.VectorSubcoreMesh(
        core_axis_name="c", subcore_axis_name="s",
        num_cores=_NC, num_subcores=_NS),
    scratch_types=[
        pltpu.VMEM((_IB, _M), jnp.int32),
        pltpu.VMEM((_C, _DIM), jnp.float32),
        pltpu.VMEM((_C, _DIM), jnp.float32),
        pltpu.SemaphoreType.DMA,
        pltpu.SemaphoreType.DMA,
        pltpu.SemaphoreType.DMA,
        pltpu.SemaphoreType.DMA,
    ],
    compiler_params=pltpu.CompilerParams(use_tc_tiling_on_sc=False),
)
def _sc_lookup(idx_hbm, table_hbm, out_hbm, idx_v, buf0, buf1,
               gsem0, gsem1, wsem0, wsem1):
    wid = lax.axis_index("s") * _NC + lax.axis_index("c")
    irow0 = wid * _IDX_ROWS_W
    orow0 = wid * _PER_W

    bufs = (buf0, buf1)
    gsems = (gsem0, gsem1)
    wsems = (wsem0, wsem1)

    def step(g, b, drain):
        @pl.when(g % _REFRESH == 0)
        def _():
            pltpu.sync_copy(
                idx_hbm.at[pl.ds(irow0 + (g // _REFRESH) * _IB, _IB)], idx_v)
        if drain is not None:
            @pl.when(drain)
            def _():
                pltpu.make_async_copy(
                    out_hbm.at[pl.ds(orow0, _C)], bufs[b], wsems[b]).wait()
        loc = (g % _REFRESH) * _K
        copies = [
            pltpu.async_copy(
                table_hbm.at[idx_v.at[loc + j]],
                bufs[b].at[pl.ds(j * _M, _M)],
                gsems[b],
            )
            for j in range(_K)
        ]
        for cp in copies:
            cp.wait()
        pltpu.async_copy(bufs[b], out_hbm.at[pl.ds(orow0 + g * _C, _C)],
                         wsems[b])

    step(0, 0, None)

    def body(t, carry):
        step(1 + 2 * t, 1, t >= 1)
        step(2 + 2 * t, 0, jnp.bool_(True))
        return carry

    lax.fori_loop(0, (_STEPS - 1) // 2, body, 0)

    pltpu.make_async_copy(out_hbm.at[pl.ds(orow0, _C)], buf1, wsem1).wait()
    pltpu.make_async_copy(out_hbm.at[pl.ds(orow0, _C)], buf0, wsem0).wait()


def kernel(edge_attr, bond_embedding):
    idx = edge_attr.astype(jnp.int32).reshape(_N // _M, _M)
    out = _sc_lookup(idx, bond_embedding)
    return out.reshape(_E, _F, _DIM)


# speedup vs baseline: 2.7183x; 2.7183x over previous
import functools

import jax
import jax.numpy as jnp
from jax import lax
from jax.experimental import pallas as pl
from jax.experimental.pallas import tpu as pltpu
from jax.experimental.pallas import tpu_sc as plsc

_E = 800000
_F = 3
_DIM = 32
_N = _E * _F
_NC = 2
_NS = 16
_NW = _NC * _NS
_PER_W = _N // _NW
_C = 1000
_STEPS = _PER_W // _C
_GROUPS = (_C + 15) // 16
_REFRESH = 25
_IB = _REFRESH * _C
_L = 16


@functools.partial(
    pl.kernel,
    out_type=jax.ShapeDtypeStruct((_N * _DIM,), jnp.float32),
    mesh=plsc.VectorSubcoreMesh(
        core_axis_name="c", subcore_axis_name="s",
        num_cores=_NC, num_subcores=_NS),
    scratch_types=[
        pltpu.VMEM((_DIM * 10,), jnp.float32),
        pltpu.VMEM((_IB,), jnp.int32),
        pltpu.VMEM((_C * _DIM,), jnp.float32),
        pltpu.VMEM((_C * _DIM,), jnp.float32),
        pltpu.SemaphoreType.DMA,
        pltpu.SemaphoreType.DMA,
    ],
    compiler_params=pltpu.CompilerParams(
        use_tc_tiling_on_sc=False, needs_layout_passes=False),
)
def _sc_lookup(idx_hbm, table_hbm, out_hbm, table_v, idx_v, buf0, buf1,
               wsem0, wsem1):
    wid = lax.axis_index("s") * _NC + lax.axis_index("c")
    row0 = wid * _PER_W

    bufs = (buf0, buf1)
    wsems = (wsem0, wsem1)

    pltpu.sync_copy(table_hbm, table_v)
    viota = lax.iota(jnp.int32, _L)

    def step(g, b, drain):
        @pl.when(g % _REFRESH == 0)
        def _():
            pltpu.sync_copy(
                idx_hbm.at[pl.ds(row0 + (g // _REFRESH) * _IB, _IB)], idx_v)
        if drain is not None:
            @pl.when(drain)
            def _():
                pltpu.make_async_copy(
                    out_hbm.at[pl.ds(0, _C * _DIM)], bufs[b], wsems[b]).wait()
        goff = (g % _REFRESH) * _C

        def body2(j, carry):
            base = jnp.minimum(j * _L, _C - _L)
            idxv = idx_v[pl.ds(goff + base, _L)]
            rowbase = idxv * _DIM
            sbase = base * _DIM + viota * _DIM
            for c in range(_DIM):
                colp = (viota + c) & (_DIM - 1)
                val = plsc.load_gather(table_v, [rowbase + colp])
                plsc.store_scatter(bufs[b], [sbase + colp], val)
            return carry

        lax.fori_loop(0, _GROUPS, body2, 0)
        pltpu.async_copy(
            bufs[b],
            out_hbm.at[pl.ds((row0 + g * _C) * _DIM, _C * _DIM)],
            wsems[b])

    step(0, 0, None)

    def pair(t, carry):
        step(1 + 2 * t, 1, t >= 1)
        step(2 + 2 * t, 0, jnp.bool_(True))
        return carry

    lax.fori_loop(0, (_STEPS - 1) // 2, pair, 0)

    pltpu.make_async_copy(out_hbm.at[pl.ds(0, _C * _DIM)], buf1, wsem1).wait()
    pltpu.make_async_copy(out_hbm.at[pl.ds(0, _C * _DIM)], buf0, wsem0).wait()


def kernel(edge_attr, bond_embedding):
    idx = edge_attr.astype(jnp.int32).reshape(_N)
    table = bond_embedding.reshape(_DIM * 10)
    out = _sc_lookup(idx, table)
    return out.reshape(_E, _F, _DIM)


# speedup vs baseline: 4.4098x; 1.6223x over previous
import functools

import jax
import jax.numpy as jnp
from jax import lax
from jax.experimental import pallas as pl
from jax.experimental.pallas import tpu as pltpu
from jax.experimental.pallas import tpu_sc as plsc

_E = 800000
_F = 3
_DIM = 32
_N = _E * _F
_NC = 2
_NS = 16
_NW = _NC * _NS
_EB = _E // 128
_BPC = 2
_CHUNKS = _EB // _BPC
_GMAX = -(-_CHUNKS // _NW)
_TPAD = 33
_L = 16
_CIDX = _BPC * 128 * _F
_CBUF = _F * 4 * _BPC * 8 * 128


@functools.partial(
    pl.kernel,
    out_type=jax.ShapeDtypeStruct((_N * _DIM,), jnp.float32),
    mesh=plsc.VectorSubcoreMesh(
        core_axis_name="c", subcore_axis_name="s",
        num_cores=_NC, num_subcores=_NS),
    scratch_types=[
        pltpu.VMEM((10 * _TPAD,), jnp.float32),
        pltpu.VMEM((_CIDX,), jnp.int32),
        pltpu.VMEM((_CBUF,), jnp.float32),
        pltpu.VMEM((_CBUF,), jnp.float32),
        pltpu.SemaphoreType.DMA,
        pltpu.SemaphoreType.DMA,
    ],
    compiler_params=pltpu.CompilerParams(
        use_tc_tiling_on_sc=False, needs_layout_passes=False),
)
def _sc_lookup(idx_hbm, table_hbm, out_hbm, table_v, idx_v, buf0, buf1,
               wsem0, wsem1):
    wid = lax.axis_index("s") * _NC + lax.axis_index("c")

    bufs = (buf0, buf1)
    wsems = (wsem0, wsem1)

    pltpu.sync_copy(table_hbm, table_v)
    viota = lax.iota(jnp.int32, _L)
    iota3 = viota * _F

    def step(g, b, drain):
        cid = g * _NW + wid
        live = cid < _CHUNKS
        if drain is not None:
            @pl.when(drain)
            def _():
                pltpu.make_async_copy(
                    out_hbm.at[pl.ds(0, _CBUF)], bufs[b], wsems[b]).wait()

        @pl.when(live)
        def _():
            pltpu.sync_copy(idx_hbm.at[pl.ds(cid * _CIDX, _CIDX)], idx_v)

            def group(t, carry):
                ebl = t // 8
                el0 = (t % 8) * _L
                base3 = (ebl * 128 + el0) * _F
                for f in range(_F):
                    idxf = plsc.load_gather(idx_v, [iota3 + (base3 + f)])
                    rb = idxf * _TPAD
                    for c in range(_DIM):
                        val = plsc.load_gather(table_v, [rb + c])
                        off = ((f * 4 + c // 8) * _BPC) * 1024 \
                            + (c % 8) * 128 + ebl * 1024 + el0
                        bufs[b][pl.ds(off, _L)] = val
                return carry

            lax.fori_loop(0, _BPC * 8, group, 0)
            for t in range(_F * 4):
                pltpu.async_copy(
                    bufs[b].at[pl.ds(t * (_BPC * 1024), _BPC * 1024)],
                    out_hbm.at[pl.ds((t * _EB + cid * _BPC) * 1024,
                                     _BPC * 1024)],
                    wsems[b])

    step(0, 0, None)

    def pair(t, carry):
        step(1 + 2 * t, 1, t >= 1)
        step(2 + 2 * t, 0, jnp.bool_(True))
        return carry

    lax.fori_loop(0, (_GMAX - 2) // 2, pair, 0)
    step(_GMAX - 1, 1, jnp.bool_(True))

    pltpu.make_async_copy(out_hbm.at[pl.ds(0, _CBUF)], buf0, wsem0).wait()

    @pl.when((_GMAX - 1) * _NW + wid < _CHUNKS)
    def _():
        pltpu.make_async_copy(
            out_hbm.at[pl.ds(0, _CBUF)], buf1, wsem1).wait()


def kernel(edge_attr, bond_embedding):
    idx = edge_attr.astype(jnp.int32).reshape(_N)
    table = jnp.pad(bond_embedding, ((0, 0), (0, _TPAD - _DIM))).reshape(
        10 * _TPAD)
    flat = _sc_lookup(idx, table)
    out5 = flat.reshape(_F, 4, _EB, 8, 128)
    return out5.transpose(2, 4, 0, 1, 3).reshape(_E, _F, _DIM)


# speedup vs baseline: 18.3725x; 4.1663x over previous
import functools

import jax
import jax.numpy as jnp
from jax import lax
from jax.experimental import pallas as pl
from jax.experimental.pallas import tpu as pltpu
from jax.experimental.pallas import tpu_sc as plsc

_E = 800000
_F = 3
_DIM = 32
_N = _E * _F
_NC = 2
_NS = 16
_NW = _NC * _NS
_EB = _E // 128
_BPC = 2
_CHUNKS = _EB // _BPC
_GMAX = -(-_CHUNKS // _NW)
_TPAD = 33
_L = 16
_CIDX = _BPC * 128 * _F
_CBUF = _F * 4 * _BPC * 8 * 128


@functools.partial(
    pl.kernel,
    out_type=jax.ShapeDtypeStruct((_N * _DIM,), jnp.float32),
    mesh=plsc.VectorSubcoreMesh(
        core_axis_name="c", subcore_axis_name="s",
        num_cores=_NC, num_subcores=_NS),
    scratch_types=[
        pltpu.VMEM((10 * _TPAD,), jnp.float32),
        pltpu.VMEM((_CIDX,), jnp.int32),
        pltpu.VMEM((_CBUF,), jnp.float32),
        pltpu.VMEM((_CBUF,), jnp.float32),
        pltpu.SemaphoreType.DMA,
        pltpu.SemaphoreType.DMA,
    ],
    compiler_params=pltpu.CompilerParams(
        use_tc_tiling_on_sc=False, needs_layout_passes=False),
)
def _sc_lookup(idx0_hbm, idx1_hbm, idx2_hbm, table_hbm, out_hbm,
               table_v, idx_v, buf0, buf1, wsem0, wsem1):
    wid = lax.axis_index("s") * _NC + lax.axis_index("c")

    bufs = (buf0, buf1)
    wsems = (wsem0, wsem1)

    pltpu.sync_copy(table_hbm, table_v)
    idx_hbms = (idx0_hbm, idx1_hbm, idx2_hbm)
    _EPC = _BPC * 128

    def step(g, b, drain):
        cid = g * _NW + wid
        live = cid < _CHUNKS
        if drain is not None:
            @pl.when(drain)
            def _():
                pltpu.make_async_copy(
                    out_hbm.at[pl.ds(0, _CBUF)], bufs[b], wsems[b]).wait()

        @pl.when(live)
        def _():
            for f in range(_F):
                pltpu.sync_copy(
                    idx_hbms[f].at[pl.ds(cid * _EPC, _EPC)],
                    idx_v.at[pl.ds(f * _EPC, _EPC)])

            def group(t, carry):
                ebl = t // 8
                el0 = (t % 8) * _L
                ebase = ebl * 128 + el0
                for f in range(_F):
                    idxf = idx_v[pl.ds(f * _EPC + ebase, _L)]
                    rb = idxf * _TPAD
                    for c in range(_DIM):
                        val = plsc.load_gather(table_v, [rb + c])
                        off = ((f * 4 + c // 8) * _BPC) * 1024 \
                            + (c % 8) * 128 + ebl * 1024 + el0
                        bufs[b][pl.ds(off, _L)] = val
                return carry

            lax.fori_loop(0, _BPC * 8, group, 0)
            for t in range(_F * 4):
                pltpu.async_copy(
                    bufs[b].at[pl.ds(t * (_BPC * 1024), _BPC * 1024)],
                    out_hbm.at[pl.ds((t * _EB + cid * _BPC) * 1024,
                                     _BPC * 1024)],
                    wsems[b])

    step(0, 0, None)

    def pair(t, carry):
        step(1 + 2 * t, 1, t >= 1)
        step(2 + 2 * t, 0, jnp.bool_(True))
        return carry

    lax.fori_loop(0, (_GMAX - 2) // 2, pair, 0)
    step(_GMAX - 1, 1, jnp.bool_(True))

    pltpu.make_async_copy(out_hbm.at[pl.ds(0, _CBUF)], buf0, wsem0).wait()

    @pl.when((_GMAX - 1) * _NW + wid < _CHUNKS)
    def _():
        pltpu.make_async_copy(
            out_hbm.at[pl.ds(0, _CBUF)], buf1, wsem1).wait()


def kernel(edge_attr, bond_embedding):
    idx = edge_attr.astype(jnp.int32)
    table = jnp.pad(bond_embedding, ((0, 0), (0, _TPAD - _DIM))).reshape(
        10 * _TPAD)
    flat = _sc_lookup(idx[:, 0], idx[:, 1], idx[:, 2], table)
    out5 = flat.reshape(_F, 4, _EB, 8, 128)
    return out5.transpose(2, 4, 0, 1, 3).reshape(_E, _F, _DIM)


# speedup vs baseline: 23.3301x; 1.2698x over previous
import functools

import jax
import jax.numpy as jnp
from jax import lax
from jax.experimental import pallas as pl
from jax.experimental.pallas import tpu as pltpu
from jax.experimental.pallas import tpu_sc as plsc

_E = 800000
_F = 3
_DIM = 32
_N = _E * _F
_NC = 2
_NS = 16
_NW = _NC * _NS
_EB = _E // 128
_BPC = 2
_CHUNKS = _EB // _BPC
_GMAX = -(-_CHUNKS // _NW)
_TPAD = 33
_L = 16
_CIDX = _BPC * 128 * _F
_CBUF = _F * 4 * _BPC * 8 * 128


@functools.partial(
    pl.kernel,
    out_type=jax.ShapeDtypeStruct((_N * _DIM,), jnp.float32),
    mesh=plsc.VectorSubcoreMesh(
        core_axis_name="c", subcore_axis_name="s",
        num_cores=_NC, num_subcores=_NS),
    scratch_types=[
        pltpu.VMEM((10 * _TPAD,), jnp.float32),
        pltpu.VMEM((_CIDX,), jnp.int32),
        pltpu.VMEM((_CIDX,), jnp.int32),
        pltpu.VMEM((_CBUF,), jnp.float32),
        pltpu.VMEM((_CBUF,), jnp.float32),
        pltpu.SemaphoreType.DMA,
        pltpu.SemaphoreType.DMA,
        pltpu.SemaphoreType.DMA,
        pltpu.SemaphoreType.DMA,
    ],
    compiler_params=pltpu.CompilerParams(
        use_tc_tiling_on_sc=False, needs_layout_passes=False),
)
def _sc_lookup(idx0_hbm, idx1_hbm, idx2_hbm, table_hbm, out_hbm,
               table_v, idx_v0, idx_v1, buf0, buf1,
               wsem0, wsem1, isem0, isem1):
    wid = lax.axis_index("s") * _NC + lax.axis_index("c")

    bufs = (buf0, buf1)
    wsems = (wsem0, wsem1)
    idx_vs = (idx_v0, idx_v1)
    isems = (isem0, isem1)

    pltpu.sync_copy(table_hbm, table_v)
    idx_hbms = (idx0_hbm, idx1_hbm, idx2_hbm)
    _EPC = _BPC * 128

    def fire_idx(g, ib):
        cid = g * _NW + wid

        @pl.when(cid < _CHUNKS)
        def _():
            for f in range(_F):
                pltpu.async_copy(
                    idx_hbms[f].at[pl.ds(cid * _EPC, _EPC)],
                    idx_vs[ib].at[pl.ds(f * _EPC, _EPC)],
                    isems[ib])

    def step(g, b, drain):
        cid = g * _NW + wid
        live = cid < _CHUNKS
        fire_idx(g + 1, 1 - b)
        if drain is not None:
            @pl.when(drain)
            def _():
                pltpu.make_async_copy(
                    out_hbm.at[pl.ds(0, _CBUF)], bufs[b], wsems[b]).wait()

        @pl.when(live)
        def _():
            pltpu.make_async_copy(
                idx0_hbm.at[pl.ds(0, _CIDX)], idx_vs[b], isems[b]).wait()
            idx_v = idx_vs[b]

            def group(t, carry):
                ebl = t // 8
                el0 = (t % 8) * _L
                ebase = ebl * 128 + el0
                for f in range(_F):
                    idxf = idx_v[pl.ds(f * _EPC + ebase, _L)]
                    rb = idxf * _TPAD
                    for c in range(_DIM):
                        val = plsc.load_gather(table_v, [rb + c])
                        off = ((f * 4 + c // 8) * _BPC) * 1024 \
                            + (c % 8) * 128 + ebl * 1024 + el0
                        bufs[b][pl.ds(off, _L)] = val
                return carry

            lax.fori_loop(0, _BPC * 8, group, 0)
            for t in range(_F * 4):
                pltpu.async_copy(
                    bufs[b].at[pl.ds(t * (_BPC * 1024), _BPC * 1024)],
                    out_hbm.at[pl.ds((t * _EB + cid * _BPC) * 1024,
                                     _BPC * 1024)],
                    wsems[b])

    fire_idx(0, 0)
    step(0, 0, None)

    def pair(t, carry):
        step(1 + 2 * t, 1, t >= 1)
        step(2 + 2 * t, 0, jnp.bool_(True))
        return carry

    lax.fori_loop(0, (_GMAX - 2) // 2, pair, 0)
    step(_GMAX - 1, 1, jnp.bool_(True))

    pltpu.make_async_copy(out_hbm.at[pl.ds(0, _CBUF)], buf0, wsem0).wait()

    @pl.when((_GMAX - 1) * _NW + wid < _CHUNKS)
    def _():
        pltpu.make_async_copy(
            out_hbm.at[pl.ds(0, _CBUF)], buf1, wsem1).wait()


def kernel(edge_attr, bond_embedding):
    idx = edge_attr.astype(jnp.int32)
    table = jnp.pad(bond_embedding, ((0, 0), (0, _TPAD - _DIM))).reshape(
        10 * _TPAD)
    flat = _sc_lookup(idx[:, 0], idx[:, 1], idx[:, 2], table)
    out5 = flat.reshape(_F, 4, _EB, 8, 128)
    return out5.transpose(2, 4, 0, 1, 3).reshape(_E, _F, _DIM)


# speedup vs baseline: 52.4204x; 2.2469x over previous
import functools

import jax
import jax.numpy as jnp
from jax import lax
from jax.experimental import pallas as pl
from jax.experimental.pallas import tpu as pltpu
from jax.experimental.pallas import tpu_sc as plsc

_E = 800000
_F = 3
_DIM = 32
_N = _E * _F
_NC = 2
_NS = 16
_NW = _NC * _NS
_EB = _E // 128
_BPC = 2
_CHUNKS = _EB // _BPC
_GMAX = -(-_CHUNKS // _NW)
_TPAD = 33
_L = 16
_CIDX = _BPC * 128 * _F
_CBUF = _F * 4 * _BPC * 8 * 128


@functools.partial(
    pl.kernel,
    out_type=jax.ShapeDtypeStruct((_N * _DIM,), jnp.float32),
    mesh=plsc.VectorSubcoreMesh(
        core_axis_name="c", subcore_axis_name="s",
        num_cores=_NC, num_subcores=_NS),
    scratch_types=[
        pltpu.VMEM((10 * _TPAD,), jnp.float32),
        pltpu.VMEM((_CIDX,), jnp.int32),
        pltpu.VMEM((_CIDX,), jnp.int32),
        pltpu.VMEM((_CBUF,), jnp.float32),
        pltpu.VMEM((_CBUF,), jnp.float32),
        pltpu.SemaphoreType.DMA,
        pltpu.SemaphoreType.DMA,
        pltpu.SemaphoreType.DMA,
        pltpu.SemaphoreType.DMA,
    ],
    compiler_params=pltpu.CompilerParams(
        use_tc_tiling_on_sc=False, needs_layout_passes=False),
)
def _sc_lookup(idx0_hbm, idx1_hbm, idx2_hbm, table_hbm, out_hbm,
               table_v, idx_v0, idx_v1, buf0, buf1,
               wsem0, wsem1, isem0, isem1):
    wid = lax.axis_index("s") * _NC + lax.axis_index("c")

    bufs = (buf0, buf1)
    wsems = (wsem0, wsem1)
    idx_vs = (idx_v0, idx_v1)
    isems = (isem0, isem1)

    pltpu.sync_copy(table_hbm, table_v)
    idx_hbms = (idx0_hbm, idx1_hbm, idx2_hbm)
    _EPC = _BPC * 128

    def fire_idx(g, ib):
        cid = g * _NW + wid

        @pl.when(cid < _CHUNKS)
        def _():
            for f in range(_F):
                pltpu.async_copy(
                    idx_hbms[f].at[pl.ds(cid * _EPC, _EPC)],
                    idx_vs[ib].at[pl.ds(f * _EPC, _EPC)],
                    isems[ib])

    def step(g, b, drain):
        cid = g * _NW + wid
        live = cid < _CHUNKS
        fire_idx(g + 1, 1 - b)
        if drain is not None:
            @pl.when(drain)
            def _():
                pltpu.make_async_copy(
                    out_hbm.at[pl.ds(0, _CBUF)], bufs[b], wsems[b]).wait()

        @pl.when(live)
        def _():
            pltpu.make_async_copy(
                idx0_hbm.at[pl.ds(0, _CIDX)], idx_vs[b], isems[b]).wait()
            idx_v = idx_vs[b]

            @plsc.parallel_loop(0, _BPC * 8)
            def _group(t):
                ebl = t // 8
                el0 = (t % 8) * _L
                ebase = ebl * 128 + el0
                for f in range(_F):
                    idxf = idx_v[pl.ds(f * _EPC + ebase, _L)]
                    rb = idxf * _TPAD
                    for c in range(_DIM):
                        val = plsc.load_gather(table_v, [rb + c])
                        off = ((f * 4 + c // 8) * _BPC) * 1024 \
                            + (c % 8) * 128 + ebl * 1024 + el0
                        bufs[b][pl.ds(off, _L)] = val
            for t in range(_F * 4):
                pltpu.async_copy(
                    bufs[b].at[pl.ds(t * (_BPC * 1024), _BPC * 1024)],
                    out_hbm.at[pl.ds((t * _EB + cid * _BPC) * 1024,
                                     _BPC * 1024)],
                    wsems[b])

    fire_idx(0, 0)
    step(0, 0, None)

    def pair(t, carry):
        step(1 + 2 * t, 1, t >= 1)
        step(2 + 2 * t, 0, jnp.bool_(True))
        return carry

    lax.fori_loop(0, (_GMAX - 2) // 2, pair, 0)
    step(_GMAX - 1, 1, jnp.bool_(True))

    pltpu.make_async_copy(out_hbm.at[pl.ds(0, _CBUF)], buf0, wsem0).wait()

    @pl.when((_GMAX - 1) * _NW + wid < _CHUNKS)
    def _():
        pltpu.make_async_copy(
            out_hbm.at[pl.ds(0, _CBUF)], buf1, wsem1).wait()


def kernel(edge_attr, bond_embedding):
    idx = edge_attr.astype(jnp.int32)
    table = jnp.pad(bond_embedding, ((0, 0), (0, _TPAD - _DIM))).reshape(
        10 * _TPAD)
    flat = _sc_lookup(idx[:, 0], idx[:, 1], idx[:, 2], table)
    out5 = flat.reshape(_F, 4, _EB, 8, 128)
    return out5.transpose(2, 4, 0, 1, 3).reshape(_E, _F, _DIM)
